# baseline (speedup 1.0000x reference)
import functools

import jax
import jax.numpy as jnp
from jax.experimental import pallas as pl
from jax.experimental.pallas import tpu as pltpu

_NSTREAM = 2
_BM = 200


def _gcn_body(eps_ref, bias_ref, x_ref, w_ref, *refs, bm, n):
    adj_refs = refs[:_NSTREAM]
    out_ref = refs[_NSTREAM]
    sup_ref = refs[_NSTREAM + 1]
    i = pl.program_id(0)

    sup_ref[...] = jnp.dot(x_ref[...], w_ref[...],
                           preferred_element_type=jnp.float32)

    row = jax.lax.broadcasted_iota(jnp.int32, (bm, n), 0)
    col = jax.lax.broadcasted_iota(jnp.int32, (bm, n), 1)
    sup = sup_ref[...]
    scale = eps_ref[0, 0] - 1.0
    for k in range(_NSTREAM):
        a = adj_refs[k][...]
        acc = jnp.dot(a, sup, preferred_element_type=jnp.float32)
        base = (_NSTREAM * i + k) * bm
        d = jnp.sum(jnp.where(col == row + base, a, 0.0), axis=1,
                    keepdims=True)
        srows = sup_ref[pl.ds(base, bm), :]
        out_ref[k * bm:(k + 1) * bm, :] = (
            acc + bias_ref[0:1, :] + scale * d * srows)


def kernel(input, adj, weight, eps, bias):
    n, in_f = input.shape
    out_f = weight.shape[1]

    bm = _BM
    group = _NSTREAM * bm
    steps = n // group
    eps2 = eps.reshape(1, 1)
    bias2 = jnp.broadcast_to(bias.reshape(1, out_f), (8, out_f))

    def adj_spec(k):
        return pl.BlockSpec((bm, n), lambda i, k=k: (_NSTREAM * i + k, 0))

    body = functools.partial(_gcn_body, bm=bm, n=n)
    out = pl.pallas_call(
        body,
        grid=(steps,),
        in_specs=[
            pl.BlockSpec(memory_space=pltpu.SMEM),
            pl.BlockSpec((8, out_f), lambda i: (0, 0)),
            pl.BlockSpec((n, in_f), lambda i: (0, 0)),
            pl.BlockSpec((in_f, out_f), lambda i: (0, 0)),
            *[adj_spec(k) for k in range(_NSTREAM)],
        ],
        out_specs=pl.BlockSpec((group, out_f), lambda i: (i, 0)),
        out_shape=jax.ShapeDtypeStruct((n, out_f), jnp.float32),
        scratch_shapes=[pltpu.VMEM((n, out_f), jnp.float32)],
        compiler_params=pltpu.CompilerParams(
            dimension_semantics=("parallel",)),
    )(eps2, bias2, input, weight, *([adj] * _NSTREAM))
    return out


# speedup vs baseline: 1.0686x; 1.0686x over previous
import functools

import jax
import jax.numpy as jnp
from jax.experimental import pallas as pl
from jax.experimental.pallas import tpu as pltpu

_NSTREAM = 2
_BM = 200


def _gcn_body(eps_ref, bias_ref, x_ref, w_ref, *refs, bm, n):
    adj_refs = refs[:_NSTREAM]
    out_ref = refs[_NSTREAM]
    sup_ref = refs[_NSTREAM + 1]
    i = pl.program_id(0)

    @pl.when(i == 0)
    def _():
        sup_ref[...] = jnp.dot(x_ref[...], w_ref[...],
                               preferred_element_type=jnp.float32)

    row = jax.lax.broadcasted_iota(jnp.int32, (bm, n), 0)
    col = jax.lax.broadcasted_iota(jnp.int32, (bm, n), 1)
    sup = sup_ref[...]
    scale = eps_ref[0, 0] - 1.0
    for k in range(_NSTREAM):
        a = adj_refs[k][...]
        acc = jnp.dot(a, sup, preferred_element_type=jnp.float32)
        base = (_NSTREAM * i + k) * bm
        d = jnp.sum(jnp.where(col == row + base, a, 0.0), axis=1,
                    keepdims=True)
        srows = sup_ref[pl.ds(base, bm), :]
        out_ref[k * bm:(k + 1) * bm, :] = (
            acc + bias_ref[0:1, :] + scale * d * srows)


def kernel(input, adj, weight, eps, bias):
    n, in_f = input.shape
    out_f = weight.shape[1]

    bm = _BM
    group = _NSTREAM * bm
    steps = n // group
    eps2 = eps.reshape(1, 1)
    bias2 = jnp.broadcast_to(bias.reshape(1, out_f), (8, out_f))

    def adj_spec(k):
        return pl.BlockSpec((bm, n), lambda i, k=k: (_NSTREAM * i + k, 0))

    body = functools.partial(_gcn_body, bm=bm, n=n)
    out = pl.pallas_call(
        body,
        grid=(steps,),
        in_specs=[
            pl.BlockSpec(memory_space=pltpu.SMEM),
            pl.BlockSpec((8, out_f), lambda i: (0, 0)),
            pl.BlockSpec((n, in_f), lambda i: (0, 0)),
            pl.BlockSpec((in_f, out_f), lambda i: (0, 0)),
            *[adj_spec(k) for k in range(_NSTREAM)],
        ],
        out_specs=pl.BlockSpec((group, out_f), lambda i: (i, 0)),
        out_shape=jax.ShapeDtypeStruct((n, out_f), jnp.float32),
        scratch_shapes=[pltpu.VMEM((n, out_f), jnp.float32)],
    )(eps2, bias2, input, weight, *([adj] * _NSTREAM))
    return out
